# baseline (speedup 1.0000x reference)
import functools

import jax
import jax.numpy as jnp
from jax import lax
from jax.experimental import pallas as pl
from jax.experimental.pallas import tpu as pltpu
from jax.experimental.pallas import tpu_sc as plsc

B, T, D = 1, 2048, 1024
N_EXP, N_SHARED = 64, 2
N_ROUTED = N_EXP - N_SHARED
K_ROUTED = 6
D_FF = 512
E_PAD = 64
BLK = 128
NB = 160
PADROWS = NB * BLK
N_ASSIGN = T * K_ROUTED

_NEG = -1e30


def _router_body(x_ref, wgt_ref, bias_ref, tri_ref, up_ref,
                 g_ref, dest_ref, counts_ref):
    x = x_ref[...]
    logits = jnp.dot(x, wgt_ref[...], preferred_element_type=jnp.float32)
    s = 1.0 / (1.0 + jnp.exp(-logits))
    col = lax.broadcasted_iota(jnp.int32, (T, E_PAD), 1).astype(jnp.float32)
    valid_e = col < float(N_ROUTED)
    bias = bias_ref[0:1, :]
    biased = jnp.where(valid_e, s + bias, _NEG)

    M = jnp.zeros((T, E_PAD), jnp.float32)
    sels = []
    ohs = []
    for _ in range(K_ROUTED):
        m = jnp.max(biased, axis=1, keepdims=True)
        is_max = biased >= m
        idx = jnp.min(jnp.where(is_max, col, float(E_PAD)), axis=1,
                      keepdims=True)
        oh = (col == idx).astype(jnp.float32)
        sels.append(jnp.sum(s * oh, axis=1, keepdims=True))
        ohs.append(oh)
        biased = jnp.where(oh > 0.0, _NEG, biased)
        M = M + oh

    sel = jnp.concatenate(sels, axis=1)
    g = sel / (jnp.sum(sel, axis=1, keepdims=True) + 1e-20)

    cum = jnp.dot(tri_ref[...], M, preferred_element_type=jnp.float32)
    counts = jnp.sum(M, axis=0, keepdims=True)
    nb = jnp.floor((counts + float(BLK - 1)) * (1.0 / BLK))
    pstart = float(BLK) * jnp.dot(nb, up_ref[...],
                                  preferred_element_type=jnp.float32)

    base = pstart + cum
    dests = [jnp.sum(ohs[j] * base, axis=1, keepdims=True)
             for j in range(K_ROUTED)]
    dest = jnp.concatenate(dests, axis=1)

    zeros2 = jnp.zeros((T, 2), jnp.float32)
    g_ref[...] = jnp.concatenate([g, zeros2], axis=1)
    dest_ref[...] = jnp.concatenate([dest, zeros2], axis=1).astype(jnp.int32)
    counts_ref[...] = jnp.broadcast_to(counts, (8, E_PAD)).astype(jnp.int32)


def _router(x_flat, Wg, expert_bias, interpret=False):
    wgt = jnp.zeros((D, E_PAD), jnp.float32).at[:, :N_ROUTED].set(Wg.T)
    bias = jnp.zeros((8, E_PAD), jnp.float32).at[:, :N_ROUTED].set(
        expert_bias[None, :])
    tri = jnp.tril(jnp.ones((T, T), jnp.float32), -1)
    up = jnp.triu(jnp.ones((E_PAD, E_PAD), jnp.float32), 1)
    return pl.pallas_call(
        _router_body,
        out_shape=(
            jax.ShapeDtypeStruct((T, 8), jnp.float32),
            jax.ShapeDtypeStruct((T, 8), jnp.int32),
            jax.ShapeDtypeStruct((8, E_PAD), jnp.int32),
        ),
        interpret=interpret,
    )(x_flat, wgt, bias, tri, up)


def _shared_body(x_ref, w1_ref, w3_ref, w2_ref, o_ref):
    x = x_ref[...].astype(jnp.bfloat16)
    acc = jnp.zeros((x.shape[0], D), jnp.float32)
    for i in range(N_SHARED):
        h1 = jnp.dot(x, w1_ref[i].T, preferred_element_type=jnp.float32)
        h3 = jnp.dot(x, w3_ref[i].T, preferred_element_type=jnp.float32)
        h = (h1 * (1.0 / (1.0 + jnp.exp(-h1))) * h3).astype(jnp.bfloat16)
        acc = acc + jnp.dot(h, w2_ref[i].T, preferred_element_type=jnp.float32)
    o_ref[...] = acc


def _shared(x_flat, Ws1, Ws2, Ws3, interpret=False):
    blk = 512
    return pl.pallas_call(
        _shared_body,
        grid=(T // blk,),
        in_specs=[
            pl.BlockSpec((blk, D), lambda i: (i, 0)),
            pl.BlockSpec((N_SHARED, D_FF, D), lambda i: (0, 0, 0)),
            pl.BlockSpec((N_SHARED, D_FF, D), lambda i: (0, 0, 0)),
            pl.BlockSpec((N_SHARED, D, D_FF), lambda i: (0, 0, 0)),
        ],
        out_specs=pl.BlockSpec((blk, D), lambda i: (i, 0)),
        out_shape=jax.ShapeDtypeStruct((T, D), jnp.float32),
        interpret=interpret,
    )(x_flat, Ws1, Ws3, Ws2)


def _gmm_body(be_ref, bl_ref, x_ref, g_ref, w1_ref, w3_ref, w2_ref, o_ref):
    b = pl.program_id(0)
    limit = bl_ref[b]

    @pl.when(limit > 0)
    def _():
        x = x_ref[...].astype(jnp.bfloat16)
        h1 = jnp.dot(x, w1_ref[0].T, preferred_element_type=jnp.float32)
        h3 = jnp.dot(x, w3_ref[0].T, preferred_element_type=jnp.float32)
        h = (h1 * (1.0 / (1.0 + jnp.exp(-h1))) * h3).astype(jnp.bfloat16)
        y = jnp.dot(h, w2_ref[0].T, preferred_element_type=jnp.float32)
        rows = lax.broadcasted_iota(jnp.int32, (BLK, 1), 0)
        gval = jnp.where(rows < limit, g_ref[:, 0:1], 0.0)
        o_ref[...] = y * gval

    @pl.when(limit <= 0)
    def _():
        o_ref[...] = jnp.zeros((BLK, D), jnp.float32)


def _gmm(x_sorted, g_sorted, Wr1, Wr2, Wr3, block_expert, block_limit,
         interpret=False):
    def _in_idx(b, be, bl):
        return (jnp.where(bl[b] > 0, b, 0), 0)

    def _w_idx(b, be, bl):
        return (be[b], 0, 0)

    def _out_idx(b, be, bl):
        return (jnp.where(bl[b] > 0, b, NB - 1), 0)

    grid_spec = pltpu.PrefetchScalarGridSpec(
        num_scalar_prefetch=2,
        grid=(NB,),
        in_specs=[
            pl.BlockSpec((BLK, D), _in_idx),
            pl.BlockSpec((BLK, 128), _in_idx),
            pl.BlockSpec((1, D_FF, D), _w_idx),
            pl.BlockSpec((1, D_FF, D), _w_idx),
            pl.BlockSpec((1, D, D_FF), _w_idx),
        ],
        out_specs=pl.BlockSpec((BLK, D), _out_idx),
    )
    if interpret:
        grid_spec = pltpu.PrefetchScalarGridSpec(
            num_scalar_prefetch=2,
            grid=(NB,),
            in_specs=[
                pl.BlockSpec((BLK, D), lambda b, be, bl: (b, 0)),
                pl.BlockSpec((BLK, 128), lambda b, be, bl: (b, 0)),
                pl.BlockSpec((1, D_FF, D), _w_idx),
                pl.BlockSpec((1, D_FF, D), _w_idx),
                pl.BlockSpec((1, D, D_FF), _w_idx),
            ],
            out_specs=pl.BlockSpec((BLK, D), lambda b, be, bl: (b, 0)),
        )
    return pl.pallas_call(
        _gmm_body,
        grid_spec=grid_spec,
        out_shape=jax.ShapeDtypeStruct((PADROWS, D), jnp.float32),
        compiler_params=pltpu.CompilerParams(
            dimension_semantics=("arbitrary",)),
        interpret=interpret,
    )(block_expert, block_limit, x_sorted, g_sorted, Wr1, Wr3, Wr2)


NW = 32
APW = N_ASSIGN // NW
DCH = 64
NDC = APW // DCH
TPW = T // NW
CCH = 8


def _dispatch_sc_body(xpad, rowi, desti, gw, xs, gs,
                      ridx, didx, buf, gbuf, sem, sem2):
    wid = lax.axis_index("s") * 2 + lax.axis_index("c")
    pltpu.sync_copy(rowi.at[wid], ridx)
    pltpu.sync_copy(desti.at[wid], didx)
    for c in range(NDC):
        pltpu.async_copy(xpad.at[ridx.at[c]], buf, sem).wait()
        pltpu.sync_copy(gw.at[wid, c], gbuf)
        pltpu.async_copy(buf, xs.at[didx.at[c]], sem).wait()
        pltpu.async_copy(gbuf, gs.at[didx.at[c]], sem2).wait()


def _dispatch_sc(x_pad, rowi, desti, gw):
    mesh = plsc.VectorSubcoreMesh(core_axis_name="c", subcore_axis_name="s")
    f = pl.kernel(
        _dispatch_sc_body,
        mesh=mesh,
        out_type=(
            jax.ShapeDtypeStruct((PADROWS, D), jnp.float32),
            jax.ShapeDtypeStruct((PADROWS, 128), jnp.float32),
        ),
        scratch_types=[
            pltpu.VMEM((NDC, DCH), jnp.int32),
            pltpu.VMEM((NDC, DCH), jnp.int32),
            pltpu.VMEM((DCH, D), jnp.float32),
            pltpu.VMEM((DCH, 128), jnp.float32),
            pltpu.SemaphoreType.DMA,
            pltpu.SemaphoreType.DMA,
        ],
    )
    return f(x_pad, rowi, desti, gw)


def _combine_sc_body(yg, desti, shared, y, dref, buf, shbuf, obuf, sem):
    wid = lax.axis_index("s") * 2 + lax.axis_index("c")
    pltpu.sync_copy(desti.at[wid], dref)
    for c in range(TPW // CCH):
        tok0 = wid * TPW + c * CCH
        gcp = pltpu.async_copy(yg.at[dref.at[c]], buf, sem)
        pltpu.sync_copy(shared.at[pl.ds(tok0, CCH)], shbuf)
        gcp.wait()
        for tt in range(CCH):
            def body_k(k, carry):
                sl = pl.ds(k * 16, 16)
                acc = shbuf[tt, sl]
                for j in range(K_ROUTED):
                    acc = acc + buf[tt * 8 + j, sl]
                obuf[tt, sl] = acc
                return carry
            lax.fori_loop(0, D // 16, body_k, 0)
        pltpu.sync_copy(obuf, y.at[pl.ds(tok0, CCH)])


def _combine_sc(yg_sorted, desti2, shared_out):
    mesh = plsc.VectorSubcoreMesh(core_axis_name="c", subcore_axis_name="s")
    f = pl.kernel(
        _combine_sc_body,
        mesh=mesh,
        out_type=jax.ShapeDtypeStruct((T, D), jnp.float32),
        scratch_types=[
            pltpu.VMEM((TPW // CCH, CCH * 8), jnp.int32),
            pltpu.VMEM((CCH * 8, D), jnp.float32),
            pltpu.VMEM((CCH, D), jnp.float32),
            pltpu.VMEM((CCH, D), jnp.float32),
            pltpu.SemaphoreType.DMA,
        ],
    )
    return f(yg_sorted, desti2, shared_out)


def _dispatch_jnp(x_pad, row_flat, dest_flat, g_flat):
    rows_sorted = jnp.full((PADROWS,), T, jnp.int32).at[dest_flat].set(row_flat)
    x_sorted = x_pad[rows_sorted]
    g_sorted = jnp.zeros((PADROWS, 16), jnp.float32).at[dest_flat, 0].set(g_flat)
    return x_sorted, g_sorted


def _combine_jnp(yg_sorted, dest6, shared_out):
    contrib = yg_sorted[dest6.reshape(-1)].reshape(T, K_ROUTED, D)
    return shared_out + jnp.sum(contrib, axis=1)


def _moe(x, Wg, expert_bias, Ws1, Ws2, Ws3, Wr1, Wr2, Wr3, interpret=False):
    x_flat = x.reshape(T, D)
    g8, dest8, counts8 = _router(x_flat, Wg, expert_bias, interpret=interpret)

    counts = counts8[0, :N_ROUTED]
    nb = (counts + (BLK - 1)) // BLK
    cumb = jnp.cumsum(nb)
    bidx = jnp.arange(NB, dtype=jnp.int32)
    block_expert = jnp.minimum(
        jnp.sum(cumb[None, :] <= bidx[:, None], axis=1), N_ROUTED - 1
    ).astype(jnp.int32)
    pstart = (cumb - nb) * BLK
    bexp_limit = pstart[block_expert] + counts[block_expert] - bidx * BLK
    block_limit = jnp.clip(bexp_limit, 0, BLK).astype(jnp.int32)

    dest6 = dest8[:, :K_ROUTED]
    x_pad = jnp.concatenate([x_flat, jnp.zeros((1, D), jnp.float32)], axis=0)

    if interpret:
        dest_flat = dest6.reshape(-1)
        row_flat = jnp.repeat(jnp.arange(T, dtype=jnp.int32), K_ROUTED)
        g_flat = g8[:, :K_ROUTED].reshape(-1)
        x_sorted, g_sorted = _dispatch_jnp(x_pad, row_flat, dest_flat, g_flat)
        shared_out = _shared(x_flat, Ws1, Ws2, Ws3, interpret=interpret)
        yg_sorted = _gmm(x_sorted, g_sorted, Wr1, Wr2, Wr3,
                         block_expert, block_limit, interpret=interpret)
        y = _combine_jnp(yg_sorted, dest6, shared_out)
        return y.reshape(B, T, D), jnp.asarray(0.0, dtype=jnp.float32)

    rowi = jnp.broadcast_to(
        jnp.arange(T, dtype=jnp.int32)[:, None], (T, K_ROUTED)
    ).reshape(NW, NDC, DCH)
    desti = dest6.reshape(NW, NDC, DCH)
    g_flat = g8[:, :K_ROUTED].reshape(-1)
    gw = jnp.zeros((N_ASSIGN, 128), jnp.float32).at[:, 0].set(g_flat)
    gw = gw.reshape(NW, NDC, DCH, 128)
    x_sorted, g_sorted = _dispatch_sc(x_pad, rowi, desti, gw)
    shared_out = _shared(x_flat, Ws1.astype(jnp.bfloat16),
                         Ws2.astype(jnp.bfloat16), Ws3.astype(jnp.bfloat16),
                         interpret=interpret)
    yg_sorted = _gmm(x_sorted, g_sorted, Wr1.astype(jnp.bfloat16),
                     Wr2.astype(jnp.bfloat16), Wr3.astype(jnp.bfloat16),
                     block_expert, block_limit, interpret=interpret)
    desti2 = dest8.reshape(NW, TPW // CCH, CCH * 8)
    y = _combine_sc(yg_sorted, desti2, shared_out)
    aux_loss = jnp.asarray(0.0, dtype=jnp.float32)
    return y.reshape(B, T, D), aux_loss


def kernel(x, Wg, expert_bias, Ws1, Ws2, Ws3, Wr1, Wr2, Wr3):
    return _moe(x, Wg, expert_bias, Ws1, Ws2, Ws3, Wr1, Wr2, Wr3)


# speedup vs baseline: 1.2022x; 1.2022x over previous
import functools

import jax
import jax.numpy as jnp
from jax import lax
from jax.experimental import pallas as pl
from jax.experimental.pallas import tpu as pltpu
from jax.experimental.pallas import tpu_sc as plsc

B, T, D = 1, 2048, 1024
N_EXP, N_SHARED = 64, 2
N_ROUTED = N_EXP - N_SHARED
K_ROUTED = 6
D_FF = 512
E_PAD = 64
BLK = 128
NB = 160
PADROWS = NB * BLK
N_ASSIGN = T * K_ROUTED

_NEG = -1e30


def _router_body(x_ref, wgt_ref, bias_ref, tri_ref, up_ref,
                 g_ref, dest_ref, counts_ref):
    x = x_ref[...]
    logits = jnp.dot(x, wgt_ref[...], preferred_element_type=jnp.float32)
    s = 1.0 / (1.0 + jnp.exp(-logits))
    col = lax.broadcasted_iota(jnp.int32, (T, E_PAD), 1).astype(jnp.float32)
    valid_e = col < float(N_ROUTED)
    bias = bias_ref[0:1, :]
    biased = jnp.where(valid_e, s + bias, _NEG)

    M = jnp.zeros((T, E_PAD), jnp.float32)
    sels = []
    ohs = []
    for _ in range(K_ROUTED):
        m = jnp.max(biased, axis=1, keepdims=True)
        is_max = biased >= m
        idx = jnp.min(jnp.where(is_max, col, float(E_PAD)), axis=1,
                      keepdims=True)
        oh = (col == idx).astype(jnp.float32)
        sels.append(jnp.sum(s * oh, axis=1, keepdims=True))
        ohs.append(oh)
        biased = jnp.where(oh > 0.0, _NEG, biased)
        M = M + oh

    sel = jnp.concatenate(sels, axis=1)
    g = sel / (jnp.sum(sel, axis=1, keepdims=True) + 1e-20)

    cum = jnp.dot(tri_ref[...], M, preferred_element_type=jnp.float32)
    counts = jnp.sum(M, axis=0, keepdims=True)
    nb = jnp.floor((counts + float(BLK - 1)) * (1.0 / BLK))
    pstart = float(BLK) * jnp.dot(nb, up_ref[...],
                                  preferred_element_type=jnp.float32)

    base = pstart + cum
    dests = [jnp.sum(ohs[j] * base, axis=1, keepdims=True)
             for j in range(K_ROUTED)]
    dest = jnp.concatenate(dests, axis=1)

    zeros2 = jnp.zeros((T, 2), jnp.float32)
    g_ref[...] = jnp.concatenate([g, zeros2], axis=1)
    dest_ref[...] = jnp.concatenate([dest, zeros2], axis=1).astype(jnp.int32)
    counts_ref[...] = jnp.broadcast_to(counts, (8, E_PAD)).astype(jnp.int32)


def _router(x_flat, Wg, expert_bias, interpret=False):
    wgt = jnp.zeros((D, E_PAD), jnp.float32).at[:, :N_ROUTED].set(Wg.T)
    bias = jnp.zeros((8, E_PAD), jnp.float32).at[:, :N_ROUTED].set(
        expert_bias[None, :])
    tri = jnp.tril(jnp.ones((T, T), jnp.float32), -1)
    up = jnp.triu(jnp.ones((E_PAD, E_PAD), jnp.float32), 1)
    return pl.pallas_call(
        _router_body,
        out_shape=(
            jax.ShapeDtypeStruct((T, 8), jnp.float32),
            jax.ShapeDtypeStruct((T, 8), jnp.int32),
            jax.ShapeDtypeStruct((8, E_PAD), jnp.int32),
        ),
        interpret=interpret,
    )(x_flat, wgt, bias, tri, up)


def _shared_body(x_ref, w1_ref, w3_ref, w2_ref, o_ref):
    x = x_ref[...].astype(jnp.bfloat16)
    acc = jnp.zeros((x.shape[0], D), jnp.float32)
    for i in range(N_SHARED):
        h1 = jnp.dot(x, w1_ref[i].astype(jnp.bfloat16).T,
                     preferred_element_type=jnp.float32)
        h3 = jnp.dot(x, w3_ref[i].astype(jnp.bfloat16).T,
                     preferred_element_type=jnp.float32)
        h = (h1 * (1.0 / (1.0 + jnp.exp(-h1))) * h3).astype(jnp.bfloat16)
        acc = acc + jnp.dot(h, w2_ref[i].astype(jnp.bfloat16).T,
                            preferred_element_type=jnp.float32)
    o_ref[...] = acc


def _shared(x_flat, Ws1, Ws2, Ws3, interpret=False):
    blk = 512
    return pl.pallas_call(
        _shared_body,
        grid=(T // blk,),
        in_specs=[
            pl.BlockSpec((blk, D), lambda i: (i, 0)),
            pl.BlockSpec((N_SHARED, D_FF, D), lambda i: (0, 0, 0)),
            pl.BlockSpec((N_SHARED, D_FF, D), lambda i: (0, 0, 0)),
            pl.BlockSpec((N_SHARED, D, D_FF), lambda i: (0, 0, 0)),
        ],
        out_specs=pl.BlockSpec((blk, D), lambda i: (i, 0)),
        out_shape=jax.ShapeDtypeStruct((T, D), jnp.float32),
        interpret=interpret,
    )(x_flat, Ws1, Ws3, Ws2)


def _gmm_body(be_ref, bl_ref, x_ref, g_ref, w1_ref, w3_ref, w2_ref, o_ref):
    b = pl.program_id(0)
    limit = bl_ref[b]

    @pl.when(limit > 0)
    def _():
        x = x_ref[...].astype(jnp.bfloat16)
        h1 = jnp.dot(x, w1_ref[0].astype(jnp.bfloat16).T,
                     preferred_element_type=jnp.float32)
        h3 = jnp.dot(x, w3_ref[0].astype(jnp.bfloat16).T,
                     preferred_element_type=jnp.float32)
        h = (h1 * (1.0 / (1.0 + jnp.exp(-h1))) * h3).astype(jnp.bfloat16)
        y = jnp.dot(h, w2_ref[0].astype(jnp.bfloat16).T,
                    preferred_element_type=jnp.float32)
        rows = lax.broadcasted_iota(jnp.int32, (BLK, 1), 0)
        gval = jnp.where(rows < limit, g_ref[:, 0:1], 0.0)
        o_ref[...] = y * gval

    @pl.when(limit <= 0)
    def _():
        o_ref[...] = jnp.zeros((BLK, D), jnp.float32)


def _gmm(x_sorted, g_sorted, Wr1, Wr2, Wr3, block_expert, block_limit,
         interpret=False):
    def _in_idx(b, be, bl):
        return (jnp.where(bl[b] > 0, b, 0), 0)

    def _w_idx(b, be, bl):
        return (be[b], 0, 0)

    def _out_idx(b, be, bl):
        return (jnp.where(bl[b] > 0, b, NB - 1), 0)

    grid_spec = pltpu.PrefetchScalarGridSpec(
        num_scalar_prefetch=2,
        grid=(NB,),
        in_specs=[
            pl.BlockSpec((BLK, D), _in_idx),
            pl.BlockSpec((BLK, 128), _in_idx),
            pl.BlockSpec((1, D_FF, D), _w_idx),
            pl.BlockSpec((1, D_FF, D), _w_idx),
            pl.BlockSpec((1, D, D_FF), _w_idx),
        ],
        out_specs=pl.BlockSpec((BLK, D), _out_idx),
    )
    if interpret:
        grid_spec = pltpu.PrefetchScalarGridSpec(
            num_scalar_prefetch=2,
            grid=(NB,),
            in_specs=[
                pl.BlockSpec((BLK, D), lambda b, be, bl: (b, 0)),
                pl.BlockSpec((BLK, 128), lambda b, be, bl: (b, 0)),
                pl.BlockSpec((1, D_FF, D), _w_idx),
                pl.BlockSpec((1, D_FF, D), _w_idx),
                pl.BlockSpec((1, D, D_FF), _w_idx),
            ],
            out_specs=pl.BlockSpec((BLK, D), lambda b, be, bl: (b, 0)),
        )
    return pl.pallas_call(
        _gmm_body,
        grid_spec=grid_spec,
        out_shape=jax.ShapeDtypeStruct((PADROWS, D), jnp.float32),
        compiler_params=pltpu.CompilerParams(
            dimension_semantics=("arbitrary",)),
        interpret=interpret,
    )(block_expert, block_limit, x_sorted, g_sorted, Wr1, Wr3, Wr2)


NW = 32
APW = N_ASSIGN // NW
DCH = 64
NDC = APW // DCH
TPW = T // NW
CCH = 8


def _dispatch_sc_body(xpad, rowi, desti, gw, xs, gs,
                      ridx, didx, buf, gbuf, sem, sem2):
    wid = lax.axis_index("s") * 2 + lax.axis_index("c")
    pltpu.sync_copy(rowi.at[wid], ridx)
    pltpu.sync_copy(desti.at[wid], didx)
    for c in range(NDC):
        pltpu.async_copy(xpad.at[ridx.at[c]], buf, sem).wait()
        pltpu.sync_copy(gw.at[wid, c], gbuf)
        pltpu.async_copy(buf, xs.at[didx.at[c]], sem).wait()
        pltpu.async_copy(gbuf, gs.at[didx.at[c]], sem2).wait()


def _dispatch_sc(x_pad, rowi, desti, gw):
    mesh = plsc.VectorSubcoreMesh(core_axis_name="c", subcore_axis_name="s")
    f = pl.kernel(
        _dispatch_sc_body,
        mesh=mesh,
        out_type=(
            jax.ShapeDtypeStruct((PADROWS, D), jnp.float32),
            jax.ShapeDtypeStruct((PADROWS, 128), jnp.float32),
        ),
        scratch_types=[
            pltpu.VMEM((NDC, DCH), jnp.int32),
            pltpu.VMEM((NDC, DCH), jnp.int32),
            pltpu.VMEM((DCH, D), jnp.float32),
            pltpu.VMEM((DCH, 128), jnp.float32),
            pltpu.SemaphoreType.DMA,
            pltpu.SemaphoreType.DMA,
        ],
    )
    return f(x_pad, rowi, desti, gw)


def _combine_sc_body(yg, desti, shared, y, dref, buf, shbuf, obuf, sem):
    wid = lax.axis_index("s") * 2 + lax.axis_index("c")
    pltpu.sync_copy(desti.at[wid], dref)
    for c in range(TPW // CCH):
        tok0 = wid * TPW + c * CCH
        gcp = pltpu.async_copy(yg.at[dref.at[c]], buf, sem)
        pltpu.sync_copy(shared.at[pl.ds(tok0, CCH)], shbuf)
        gcp.wait()
        for tt in range(CCH):
            def body_k(k, carry):
                sl = pl.ds(k * 16, 16)
                acc = shbuf[tt, sl]
                for j in range(K_ROUTED):
                    acc = acc + buf[tt * 8 + j, sl]
                obuf[tt, sl] = acc
                return carry
            lax.fori_loop(0, D // 16, body_k, 0)
        pltpu.sync_copy(obuf, y.at[pl.ds(tok0, CCH)])


def _combine_sc(yg_sorted, desti2, shared_out):
    mesh = plsc.VectorSubcoreMesh(core_axis_name="c", subcore_axis_name="s")
    f = pl.kernel(
        _combine_sc_body,
        mesh=mesh,
        out_type=jax.ShapeDtypeStruct((T, D), jnp.float32),
        scratch_types=[
            pltpu.VMEM((TPW // CCH, CCH * 8), jnp.int32),
            pltpu.VMEM((CCH * 8, D), jnp.float32),
            pltpu.VMEM((CCH, D), jnp.float32),
            pltpu.VMEM((CCH, D), jnp.float32),
            pltpu.SemaphoreType.DMA,
        ],
    )
    return f(yg_sorted, desti2, shared_out)


def _dispatch_jnp(x_pad, row_flat, dest_flat, g_flat):
    rows_sorted = jnp.full((PADROWS,), T, jnp.int32).at[dest_flat].set(row_flat)
    x_sorted = x_pad[rows_sorted]
    g_sorted = jnp.zeros((PADROWS, 16), jnp.float32).at[dest_flat, 0].set(g_flat)
    return x_sorted, g_sorted


def _combine_jnp(yg_sorted, dest6, shared_out):
    contrib = yg_sorted[dest6.reshape(-1)].reshape(T, K_ROUTED, D)
    return shared_out + jnp.sum(contrib, axis=1)


def _moe(x, Wg, expert_bias, Ws1, Ws2, Ws3, Wr1, Wr2, Wr3, interpret=False):
    x_flat = x.reshape(T, D)
    g8, dest8, counts8 = _router(x_flat, Wg, expert_bias, interpret=interpret)

    counts = counts8[0, :N_ROUTED]
    nb = (counts + (BLK - 1)) // BLK
    cumb = jnp.cumsum(nb)
    bidx = jnp.arange(NB, dtype=jnp.int32)
    block_expert = jnp.minimum(
        jnp.sum(cumb[None, :] <= bidx[:, None], axis=1), N_ROUTED - 1
    ).astype(jnp.int32)
    pstart = (cumb - nb) * BLK
    bexp_limit = pstart[block_expert] + counts[block_expert] - bidx * BLK
    block_limit = jnp.clip(bexp_limit, 0, BLK).astype(jnp.int32)

    dest6 = dest8[:, :K_ROUTED]
    x_pad = jnp.concatenate([x_flat, jnp.zeros((1, D), jnp.float32)], axis=0)

    if interpret:
        dest_flat = dest6.reshape(-1)
        row_flat = jnp.repeat(jnp.arange(T, dtype=jnp.int32), K_ROUTED)
        g_flat = g8[:, :K_ROUTED].reshape(-1)
        x_sorted, g_sorted = _dispatch_jnp(x_pad, row_flat, dest_flat, g_flat)
        shared_out = _shared(x_flat, Ws1, Ws2, Ws3, interpret=interpret)
        yg_sorted = _gmm(x_sorted, g_sorted, Wr1, Wr2, Wr3,
                         block_expert, block_limit, interpret=interpret)
        y = _combine_jnp(yg_sorted, dest6, shared_out)
        return y.reshape(B, T, D), jnp.asarray(0.0, dtype=jnp.float32)

    rowi = jnp.broadcast_to(
        jnp.arange(T, dtype=jnp.int32)[:, None], (T, K_ROUTED)
    ).reshape(NW, NDC, DCH)
    desti = dest6.reshape(NW, NDC, DCH)
    g_flat = g8[:, :K_ROUTED].reshape(-1)
    gw = jnp.zeros((N_ASSIGN, 128), jnp.float32).at[:, 0].set(g_flat)
    gw = gw.reshape(NW, NDC, DCH, 128)
    x_sorted, g_sorted = _dispatch_sc(x_pad, rowi, desti, gw)
    shared_out = _shared(x_flat, Ws1, Ws2, Ws3, interpret=interpret)
    yg_sorted = _gmm(x_sorted, g_sorted, Wr1, Wr2, Wr3,
                     block_expert, block_limit, interpret=interpret)
    desti2 = dest8.reshape(NW, TPW // CCH, CCH * 8)
    y = _combine_sc(yg_sorted, desti2, shared_out)
    aux_loss = jnp.asarray(0.0, dtype=jnp.float32)
    return y.reshape(B, T, D), aux_loss


def kernel(x, Wg, expert_bias, Ws1, Ws2, Ws3, Wr1, Wr2, Wr3):
    return _moe(x, Wg, expert_bias, Ws1, Ws2, Ws3, Wr1, Wr2, Wr3)


# speedup vs baseline: 1.6093x; 1.3386x over previous
import functools

import jax
import jax.numpy as jnp
from jax import lax
from jax.experimental import pallas as pl
from jax.experimental.pallas import tpu as pltpu
from jax.experimental.pallas import tpu_sc as plsc

B, T, D = 1, 2048, 1024
N_EXP, N_SHARED = 64, 2
N_ROUTED = N_EXP - N_SHARED
K_ROUTED = 6
D_FF = 512
E_PAD = 64
BLK = 128
NB = 160
PADROWS = NB * BLK
N_ASSIGN = T * K_ROUTED

_NEG = -1e30


def _router_body(x_ref, wgt_ref, bias_ref, tri_ref, up_ref,
                 g_ref, dest_ref, counts_ref):
    x = x_ref[...]
    logits = jnp.dot(x, wgt_ref[...], preferred_element_type=jnp.float32)
    s = 1.0 / (1.0 + jnp.exp(-logits))
    col = lax.broadcasted_iota(jnp.int32, (T, E_PAD), 1).astype(jnp.float32)
    valid_e = col < float(N_ROUTED)
    bias = bias_ref[0:1, :]
    biased = jnp.where(valid_e, s + bias, _NEG)

    M = jnp.zeros((T, E_PAD), jnp.float32)
    sels = []
    ohs = []
    for _ in range(K_ROUTED):
        m = jnp.max(biased, axis=1, keepdims=True)
        is_max = biased >= m
        idx = jnp.min(jnp.where(is_max, col, float(E_PAD)), axis=1,
                      keepdims=True)
        oh = (col == idx).astype(jnp.float32)
        sels.append(jnp.sum(s * oh, axis=1, keepdims=True))
        ohs.append(oh)
        biased = jnp.where(oh > 0.0, _NEG, biased)
        M = M + oh

    sel = jnp.concatenate(sels, axis=1)
    g = sel / (jnp.sum(sel, axis=1, keepdims=True) + 1e-20)

    cum = jnp.dot(tri_ref[...], M, preferred_element_type=jnp.float32)
    counts = jnp.sum(M, axis=0, keepdims=True)
    nb = jnp.floor((counts + float(BLK - 1)) * (1.0 / BLK))
    pstart = float(BLK) * jnp.dot(nb, up_ref[...],
                                  preferred_element_type=jnp.float32)

    base = pstart + cum
    dests = [jnp.sum(ohs[j] * base, axis=1, keepdims=True)
             for j in range(K_ROUTED)]
    dest = jnp.concatenate(dests, axis=1)

    zeros2 = jnp.zeros((T, 2), jnp.float32)
    g_ref[...] = jnp.concatenate([g, zeros2], axis=1)
    dest_ref[...] = jnp.concatenate([dest, zeros2], axis=1).astype(jnp.int32)
    counts_ref[...] = jnp.broadcast_to(counts, (8, E_PAD)).astype(jnp.int32)


def _router(x_flat, Wg, expert_bias, interpret=False):
    wgt = jnp.zeros((D, E_PAD), jnp.float32).at[:, :N_ROUTED].set(Wg.T)
    bias = jnp.zeros((8, E_PAD), jnp.float32).at[:, :N_ROUTED].set(
        expert_bias[None, :])
    tri = jnp.tril(jnp.ones((T, T), jnp.float32), -1)
    up = jnp.triu(jnp.ones((E_PAD, E_PAD), jnp.float32), 1)
    return pl.pallas_call(
        _router_body,
        out_shape=(
            jax.ShapeDtypeStruct((T, 8), jnp.float32),
            jax.ShapeDtypeStruct((T, 8), jnp.int32),
            jax.ShapeDtypeStruct((8, E_PAD), jnp.int32),
        ),
        interpret=interpret,
    )(x_flat, wgt, bias, tri, up)


def _shared_body(x_ref, w1_ref, w3_ref, w2_ref, o_ref):
    x = x_ref[...].astype(jnp.bfloat16)
    acc = jnp.zeros((x.shape[0], D), jnp.float32)
    for i in range(N_SHARED):
        h1 = jnp.dot(x, w1_ref[i].astype(jnp.bfloat16).T,
                     preferred_element_type=jnp.float32)
        h3 = jnp.dot(x, w3_ref[i].astype(jnp.bfloat16).T,
                     preferred_element_type=jnp.float32)
        h = (h1 * (1.0 / (1.0 + jnp.exp(-h1))) * h3).astype(jnp.bfloat16)
        acc = acc + jnp.dot(h, w2_ref[i].astype(jnp.bfloat16).T,
                            preferred_element_type=jnp.float32)
    o_ref[...] = acc


def _shared(x_flat, Ws1, Ws2, Ws3, interpret=False):
    blk = 512
    return pl.pallas_call(
        _shared_body,
        grid=(T // blk,),
        in_specs=[
            pl.BlockSpec((blk, D), lambda i: (i, 0)),
            pl.BlockSpec((N_SHARED, D_FF, D), lambda i: (0, 0, 0)),
            pl.BlockSpec((N_SHARED, D_FF, D), lambda i: (0, 0, 0)),
            pl.BlockSpec((N_SHARED, D, D_FF), lambda i: (0, 0, 0)),
        ],
        out_specs=pl.BlockSpec((blk, D), lambda i: (i, 0)),
        out_shape=jax.ShapeDtypeStruct((T, D), jnp.float32),
        interpret=interpret,
    )(x_flat, Ws1, Ws3, Ws2)


def _gmm_body(be_ref, bl_ref, x_ref, g_ref, w1_ref, w3_ref, w2_ref, o_ref):
    b = pl.program_id(0)
    limit = bl_ref[b]

    @pl.when(limit > 0)
    def _():
        x = x_ref[...].astype(jnp.bfloat16)
        h1 = jnp.dot(x, w1_ref[0].astype(jnp.bfloat16).T,
                     preferred_element_type=jnp.float32)
        h3 = jnp.dot(x, w3_ref[0].astype(jnp.bfloat16).T,
                     preferred_element_type=jnp.float32)
        h = (h1 * (1.0 / (1.0 + jnp.exp(-h1))) * h3).astype(jnp.bfloat16)
        y = jnp.dot(h, w2_ref[0].astype(jnp.bfloat16).T,
                    preferred_element_type=jnp.float32)
        rows = lax.broadcasted_iota(jnp.int32, (BLK, 1), 0)
        gval = jnp.where(rows < limit, g_ref[:, 0:1], 0.0)
        o_ref[...] = y * gval

    @pl.when(limit <= 0)
    def _():
        o_ref[...] = jnp.zeros((BLK, D), jnp.float32)


def _gmm(x_sorted, g_sorted, Wr1, Wr2, Wr3, block_expert, block_limit,
         interpret=False):
    def _in_idx(b, be, bl):
        return (jnp.where(bl[b] > 0, b, 0), 0)

    def _w_idx(b, be, bl):
        return (be[b], 0, 0)

    def _out_idx(b, be, bl):
        return (jnp.where(bl[b] > 0, b, NB - 1), 0)

    grid_spec = pltpu.PrefetchScalarGridSpec(
        num_scalar_prefetch=2,
        grid=(NB,),
        in_specs=[
            pl.BlockSpec((BLK, D), _in_idx),
            pl.BlockSpec((BLK, 128), _in_idx),
            pl.BlockSpec((1, D_FF, D), _w_idx),
            pl.BlockSpec((1, D_FF, D), _w_idx),
            pl.BlockSpec((1, D, D_FF), _w_idx),
        ],
        out_specs=pl.BlockSpec((BLK, D), _out_idx),
    )
    if interpret:
        grid_spec = pltpu.PrefetchScalarGridSpec(
            num_scalar_prefetch=2,
            grid=(NB,),
            in_specs=[
                pl.BlockSpec((BLK, D), lambda b, be, bl: (b, 0)),
                pl.BlockSpec((BLK, 128), lambda b, be, bl: (b, 0)),
                pl.BlockSpec((1, D_FF, D), _w_idx),
                pl.BlockSpec((1, D_FF, D), _w_idx),
                pl.BlockSpec((1, D, D_FF), _w_idx),
            ],
            out_specs=pl.BlockSpec((BLK, D), lambda b, be, bl: (b, 0)),
        )
    return pl.pallas_call(
        _gmm_body,
        grid_spec=grid_spec,
        out_shape=jax.ShapeDtypeStruct((PADROWS, D), jnp.float32),
        compiler_params=pltpu.CompilerParams(
            dimension_semantics=("arbitrary",)),
        interpret=interpret,
    )(block_expert, block_limit, x_sorted, g_sorted, Wr1, Wr3, Wr2)


NW = 32
APW = N_ASSIGN // NW
DCH = 48
NDC = APW // DCH
TPW = T // NW
CCH = 8


def _dispatch_sc_body(xpad, rowi, desti, gw, xs, gs,
                      ridx, didx, buf0, buf1, gbuf, semg0, semg1,
                      sems0, sems1, semgb):
    wid = lax.axis_index("s") * 2 + lax.axis_index("c")
    pltpu.sync_copy(rowi.at[wid], ridx)
    pltpu.sync_copy(desti.at[wid], didx)
    bufs = (buf0, buf1)
    semg = (semg0, semg1)
    sems = (sems0, sems1)
    gcp = [None, None]
    scp = [None, None]
    gcp[0] = pltpu.async_copy(xpad.at[ridx.at[0]], bufs[0], semg[0])
    for c in range(NDC):
        cur = c % 2
        if c + 1 < NDC:
            if scp[1 - cur] is not None:
                scp[1 - cur].wait()
            gcp[1 - cur] = pltpu.async_copy(
                xpad.at[ridx.at[c + 1]], bufs[1 - cur], semg[1 - cur])
        pltpu.sync_copy(gw.at[wid, c], gbuf)
        gcp[cur].wait()
        scp[cur] = pltpu.async_copy(bufs[cur], xs.at[didx.at[c]], sems[cur])
        pltpu.async_copy(gbuf, gs.at[didx.at[c]], semgb).wait()
    for cur in range(2):
        if scp[cur] is not None:
            scp[cur].wait()


def _dispatch_sc(x_pad, rowi, desti, gw):
    mesh = plsc.VectorSubcoreMesh(core_axis_name="c", subcore_axis_name="s")
    f = pl.kernel(
        _dispatch_sc_body,
        mesh=mesh,
        out_type=(
            jax.ShapeDtypeStruct((PADROWS, D), jnp.float32),
            jax.ShapeDtypeStruct((PADROWS, 128), jnp.float32),
        ),
        scratch_types=[
            pltpu.VMEM((NDC, DCH), jnp.int32),
            pltpu.VMEM((NDC, DCH), jnp.int32),
            pltpu.VMEM((DCH, D), jnp.float32),
            pltpu.VMEM((DCH, D), jnp.float32),
            pltpu.VMEM((DCH, 128), jnp.float32),
            pltpu.SemaphoreType.DMA,
            pltpu.SemaphoreType.DMA,
            pltpu.SemaphoreType.DMA,
            pltpu.SemaphoreType.DMA,
            pltpu.SemaphoreType.DMA,
        ],
    )
    return f(x_pad, rowi, desti, gw)


_NCC = TPW // CCH


def _combine_sc_body(yg, desti, shared, y, dref,
                     buf0, buf1, shbuf, obuf, sem0, sem1):
    wid = lax.axis_index("s") * 2 + lax.axis_index("c")
    pltpu.sync_copy(desti.at[wid], dref)
    bufs = (buf0, buf1)
    sems = (sem0, sem1)
    cps = [None, None]
    cps[0] = pltpu.async_copy(yg.at[dref.at[0]], bufs[0], sems[0])
    for c in range(_NCC):
        cur = c % 2
        if c + 1 < _NCC:
            cps[1 - cur] = pltpu.async_copy(
                yg.at[dref.at[c + 1]], bufs[1 - cur], sems[1 - cur])
        tok0 = wid * TPW + c * CCH
        pltpu.sync_copy(shared.at[pl.ds(tok0, CCH)], shbuf)
        cps[cur].wait()
        buf = bufs[cur]
        for tt in range(CCH):
            def body_k(k, carry):
                sl = pl.ds(k * 16, 16)
                acc = shbuf[tt, sl]
                for j in range(K_ROUTED):
                    acc = acc + buf[tt * K_ROUTED + j, sl]
                obuf[tt, sl] = acc
                return carry
            lax.fori_loop(0, D // 16, body_k, 0)
        pltpu.sync_copy(obuf, y.at[pl.ds(tok0, CCH)])


def _combine_sc(yg_sorted, desti2, shared_out):
    mesh = plsc.VectorSubcoreMesh(core_axis_name="c", subcore_axis_name="s")
    f = pl.kernel(
        _combine_sc_body,
        mesh=mesh,
        out_type=jax.ShapeDtypeStruct((T, D), jnp.float32),
        scratch_types=[
            pltpu.VMEM((_NCC, CCH * K_ROUTED), jnp.int32),
            pltpu.VMEM((CCH * K_ROUTED, D), jnp.float32),
            pltpu.VMEM((CCH * K_ROUTED, D), jnp.float32),
            pltpu.VMEM((CCH, D), jnp.float32),
            pltpu.VMEM((CCH, D), jnp.float32),
            pltpu.SemaphoreType.DMA,
            pltpu.SemaphoreType.DMA,
        ],
    )
    return f(yg_sorted, desti2, shared_out)


def _dispatch_jnp(x_pad, row_flat, dest_flat, g_flat):
    rows_sorted = jnp.full((PADROWS,), T, jnp.int32).at[dest_flat].set(row_flat)
    x_sorted = x_pad[rows_sorted]
    g_sorted = jnp.zeros((PADROWS, 16), jnp.float32).at[dest_flat, 0].set(g_flat)
    return x_sorted, g_sorted


def _combine_jnp(yg_sorted, dest6, shared_out):
    contrib = yg_sorted[dest6.reshape(-1)].reshape(T, K_ROUTED, D)
    return shared_out + jnp.sum(contrib, axis=1)


def _moe(x, Wg, expert_bias, Ws1, Ws2, Ws3, Wr1, Wr2, Wr3, interpret=False):
    x_flat = x.reshape(T, D)
    g8, dest8, counts8 = _router(x_flat, Wg, expert_bias, interpret=interpret)

    counts = counts8[0, :N_ROUTED]
    nb = (counts + (BLK - 1)) // BLK
    cumb = jnp.cumsum(nb)
    bidx = jnp.arange(NB, dtype=jnp.int32)
    block_expert = jnp.minimum(
        jnp.sum(cumb[None, :] <= bidx[:, None], axis=1), N_ROUTED - 1
    ).astype(jnp.int32)
    pstart = (cumb - nb) * BLK
    bexp_limit = pstart[block_expert] + counts[block_expert] - bidx * BLK
    block_limit = jnp.clip(bexp_limit, 0, BLK).astype(jnp.int32)

    dest6 = dest8[:, :K_ROUTED]
    x_pad = jnp.concatenate([x_flat, jnp.zeros((1, D), jnp.float32)], axis=0)

    if interpret:
        dest_flat = dest6.reshape(-1)
        row_flat = jnp.repeat(jnp.arange(T, dtype=jnp.int32), K_ROUTED)
        g_flat = g8[:, :K_ROUTED].reshape(-1)
        x_sorted, g_sorted = _dispatch_jnp(x_pad, row_flat, dest_flat, g_flat)
        shared_out = _shared(x_flat, Ws1, Ws2, Ws3, interpret=interpret)
        yg_sorted = _gmm(x_sorted, g_sorted, Wr1, Wr2, Wr3,
                         block_expert, block_limit, interpret=interpret)
        y = _combine_jnp(yg_sorted, dest6, shared_out)
        return y.reshape(B, T, D), jnp.asarray(0.0, dtype=jnp.float32)

    rowi = jnp.broadcast_to(
        jnp.arange(T, dtype=jnp.int32)[:, None], (T, K_ROUTED)
    ).reshape(NW, NDC, DCH)
    desti = dest6.reshape(NW, NDC, DCH)
    g_flat = g8[:, :K_ROUTED].reshape(-1)
    gw = jnp.zeros((N_ASSIGN, 128), jnp.float32).at[:, 0].set(g_flat)
    gw = gw.reshape(NW, NDC, DCH, 128)
    x_sorted, g_sorted = _dispatch_sc(x_pad, rowi, desti, gw)
    shared_out = _shared(x_flat, Ws1, Ws2, Ws3, interpret=interpret)
    yg_sorted = _gmm(x_sorted, g_sorted, Wr1, Wr2, Wr3,
                     block_expert, block_limit, interpret=interpret)
    desti2 = dest6.reshape(NW, TPW // CCH, CCH * K_ROUTED)
    y = _combine_sc(yg_sorted, desti2, shared_out)
    aux_loss = jnp.asarray(0.0, dtype=jnp.float32)
    return y.reshape(B, T, D), aux_loss


def kernel(x, Wg, expert_bias, Ws1, Ws2, Ws3, Wr1, Wr2, Wr3):
    return _moe(x, Wg, expert_bias, Ws1, Ws2, Ws3, Wr1, Wr2, Wr3)


# speedup vs baseline: 1.7099x; 1.0625x over previous
import functools

import jax
import jax.numpy as jnp
from jax import lax
from jax.experimental import pallas as pl
from jax.experimental.pallas import tpu as pltpu
from jax.experimental.pallas import tpu_sc as plsc

B, T, D = 1, 2048, 1024
N_EXP, N_SHARED = 64, 2
N_ROUTED = N_EXP - N_SHARED
K_ROUTED = 6
D_FF = 512
E_PAD = 64
BLK = 128
NB = 160
PADROWS = NB * BLK
N_ASSIGN = T * K_ROUTED

_NEG = -1e30


def _router_body(x_ref, wgt_ref, bias_ref, tri_ref, up_ref,
                 g_ref, dest_ref, counts_ref):
    x = x_ref[...]
    logits = jnp.dot(x, wgt_ref[...], preferred_element_type=jnp.float32)
    s = 1.0 / (1.0 + jnp.exp(-logits))
    col = lax.broadcasted_iota(jnp.int32, (T, E_PAD), 1).astype(jnp.float32)
    valid_e = col < float(N_ROUTED)
    bias = bias_ref[0:1, :]
    biased = jnp.where(valid_e, s + bias, _NEG)

    M = jnp.zeros((T, E_PAD), jnp.float32)
    sels = []
    ohs = []
    for _ in range(K_ROUTED):
        m = jnp.max(biased, axis=1, keepdims=True)
        is_max = biased >= m
        idx = jnp.min(jnp.where(is_max, col, float(E_PAD)), axis=1,
                      keepdims=True)
        oh = (col == idx).astype(jnp.float32)
        sels.append(jnp.sum(s * oh, axis=1, keepdims=True))
        ohs.append(oh)
        biased = jnp.where(oh > 0.0, _NEG, biased)
        M = M + oh

    sel = jnp.concatenate(sels, axis=1)
    g = sel / (jnp.sum(sel, axis=1, keepdims=True) + 1e-20)

    cum = jnp.dot(tri_ref[...], M, preferred_element_type=jnp.float32)
    counts = jnp.sum(M, axis=0, keepdims=True)
    nb = jnp.floor((counts + float(BLK - 1)) * (1.0 / BLK))
    pstart = float(BLK) * jnp.dot(nb, up_ref[...],
                                  preferred_element_type=jnp.float32)

    base = pstart + cum
    dests = [jnp.sum(ohs[j] * base, axis=1, keepdims=True)
             for j in range(K_ROUTED)]
    dest = jnp.concatenate(dests, axis=1)

    zeros2 = jnp.zeros((T, 2), jnp.float32)
    g_ref[...] = jnp.concatenate([g, zeros2], axis=1)
    dest_ref[...] = jnp.concatenate([dest, zeros2], axis=1).astype(jnp.int32)
    counts_ref[...] = jnp.broadcast_to(counts, (8, E_PAD)).astype(jnp.int32)


def _router(x_flat, Wg, expert_bias, interpret=False):
    wgt = jnp.zeros((D, E_PAD), jnp.float32).at[:, :N_ROUTED].set(Wg.T)
    bias = jnp.zeros((8, E_PAD), jnp.float32).at[:, :N_ROUTED].set(
        expert_bias[None, :])
    tri = jnp.tril(jnp.ones((T, T), jnp.float32), -1)
    up = jnp.triu(jnp.ones((E_PAD, E_PAD), jnp.float32), 1)
    return pl.pallas_call(
        _router_body,
        out_shape=(
            jax.ShapeDtypeStruct((T, 8), jnp.float32),
            jax.ShapeDtypeStruct((T, 8), jnp.int32),
            jax.ShapeDtypeStruct((8, E_PAD), jnp.int32),
        ),
        interpret=interpret,
    )(x_flat, wgt, bias, tri, up)


def _shared_body(x_ref, w1_ref, w3_ref, w2_ref, o_ref):
    x = x_ref[...].astype(jnp.bfloat16)
    acc = jnp.zeros((x.shape[0], D), jnp.float32)
    for i in range(N_SHARED):
        h1 = jnp.dot(x, w1_ref[i].astype(jnp.bfloat16).T,
                     preferred_element_type=jnp.float32)
        h3 = jnp.dot(x, w3_ref[i].astype(jnp.bfloat16).T,
                     preferred_element_type=jnp.float32)
        h = (h1 * (1.0 / (1.0 + jnp.exp(-h1))) * h3).astype(jnp.bfloat16)
        acc = acc + jnp.dot(h, w2_ref[i].astype(jnp.bfloat16).T,
                            preferred_element_type=jnp.float32)
    o_ref[...] = acc


def _shared(x_flat, Ws1, Ws2, Ws3, interpret=False):
    blk = 512
    return pl.pallas_call(
        _shared_body,
        grid=(T // blk,),
        in_specs=[
            pl.BlockSpec((blk, D), lambda i: (i, 0)),
            pl.BlockSpec((N_SHARED, D_FF, D), lambda i: (0, 0, 0)),
            pl.BlockSpec((N_SHARED, D_FF, D), lambda i: (0, 0, 0)),
            pl.BlockSpec((N_SHARED, D, D_FF), lambda i: (0, 0, 0)),
        ],
        out_specs=pl.BlockSpec((blk, D), lambda i: (i, 0)),
        out_shape=jax.ShapeDtypeStruct((T, D), jnp.float32),
        interpret=interpret,
    )(x_flat, Ws1, Ws3, Ws2)


def _gmm_body(be_ref, bl_ref, x_ref, g_ref, w1_ref, w3_ref, w2_ref, o_ref):
    b = pl.program_id(0)
    limit = bl_ref[b]

    @pl.when(limit > 0)
    def _():
        x = x_ref[...].astype(jnp.bfloat16)
        h1 = jnp.dot(x, w1_ref[0].astype(jnp.bfloat16).T,
                     preferred_element_type=jnp.float32)
        h3 = jnp.dot(x, w3_ref[0].astype(jnp.bfloat16).T,
                     preferred_element_type=jnp.float32)
        h = (h1 * (1.0 / (1.0 + jnp.exp(-h1))) * h3).astype(jnp.bfloat16)
        y = jnp.dot(h, w2_ref[0].astype(jnp.bfloat16).T,
                    preferred_element_type=jnp.float32)
        rows = lax.broadcasted_iota(jnp.int32, (BLK, 1), 0)
        gval = jnp.where(rows < limit, g_ref[:, 0:1], 0.0)
        o_ref[...] = y * gval

    @pl.when(limit <= 0)
    def _():
        o_ref[...] = jnp.zeros((BLK, D), jnp.float32)


def _gmm(x_sorted, g_sorted, Wr1, Wr2, Wr3, block_expert, block_limit,
         interpret=False):
    def _in_idx(b, be, bl):
        return (jnp.where(bl[b] > 0, b, 0), 0)

    def _w_idx(b, be, bl):
        return (be[b], 0, 0)

    def _out_idx(b, be, bl):
        return (jnp.where(bl[b] > 0, b, NB - 1), 0)

    grid_spec = pltpu.PrefetchScalarGridSpec(
        num_scalar_prefetch=2,
        grid=(NB,),
        in_specs=[
            pl.BlockSpec((BLK, D), _in_idx),
            pl.BlockSpec((BLK, 128), _in_idx),
            pl.BlockSpec((1, D_FF, D), _w_idx),
            pl.BlockSpec((1, D_FF, D), _w_idx),
            pl.BlockSpec((1, D, D_FF), _w_idx),
        ],
        out_specs=pl.BlockSpec((BLK, D), _out_idx),
    )
    if interpret:
        grid_spec = pltpu.PrefetchScalarGridSpec(
            num_scalar_prefetch=2,
            grid=(NB,),
            in_specs=[
                pl.BlockSpec((BLK, D), lambda b, be, bl: (b, 0)),
                pl.BlockSpec((BLK, 128), lambda b, be, bl: (b, 0)),
                pl.BlockSpec((1, D_FF, D), _w_idx),
                pl.BlockSpec((1, D_FF, D), _w_idx),
                pl.BlockSpec((1, D, D_FF), _w_idx),
            ],
            out_specs=pl.BlockSpec((BLK, D), lambda b, be, bl: (b, 0)),
        )
    return pl.pallas_call(
        _gmm_body,
        grid_spec=grid_spec,
        out_shape=jax.ShapeDtypeStruct((PADROWS, D), jnp.float32),
        compiler_params=pltpu.CompilerParams(
            dimension_semantics=("arbitrary",)),
        interpret=interpret,
    )(block_expert, block_limit, x_sorted, g_sorted, Wr1, Wr3, Wr2)


NW = 32
APW = N_ASSIGN // NW
DCH = 48
NDC = APW // DCH
TPW = T // NW
CCH = 8


def _dispatch_sc_body(xpad, desti, gw, xs, gs, didx, buf, gwb, sem, sem2):
    wid = lax.axis_index("s") * 2 + lax.axis_index("c")
    tok0 = wid * TPW
    pltpu.sync_copy(desti.at[wid], didx)
    pltpu.sync_copy(gw.at[wid], gwb)
    pltpu.sync_copy(xpad.at[pl.ds(tok0, TPW)], buf)
    cps = []
    for j in range(K_ROUTED):
        cps.append(pltpu.async_copy(buf, xs.at[didx.at[j]], sem))
        cps.append(pltpu.async_copy(gwb.at[j], gs.at[didx.at[j]], sem2))
    for cp in cps:
        cp.wait()


def _dispatch_sc(x_flat, desti, gw):
    mesh = plsc.VectorSubcoreMesh(core_axis_name="c", subcore_axis_name="s")
    f = pl.kernel(
        _dispatch_sc_body,
        mesh=mesh,
        out_type=(
            jax.ShapeDtypeStruct((PADROWS, D), jnp.float32),
            jax.ShapeDtypeStruct((PADROWS, 128), jnp.float32),
        ),
        scratch_types=[
            pltpu.VMEM((K_ROUTED, TPW), jnp.int32),
            pltpu.VMEM((TPW, D), jnp.float32),
            pltpu.VMEM((K_ROUTED, TPW, 128), jnp.float32),
            pltpu.SemaphoreType.DMA,
            pltpu.SemaphoreType.DMA,
        ],
    )
    return f(x_flat, desti, gw)


_NCC = TPW // CCH


def _combine_sc_body(yg, desti, shared, y, dref,
                     buf0, buf1, shbuf, obuf, sem0, sem1):
    wid = lax.axis_index("s") * 2 + lax.axis_index("c")
    pltpu.sync_copy(desti.at[wid], dref)
    bufs = (buf0, buf1)
    sems = (sem0, sem1)
    cps = [None, None]
    cps[0] = pltpu.async_copy(yg.at[dref.at[0]], bufs[0], sems[0])
    for c in range(_NCC):
        cur = c % 2
        if c + 1 < _NCC:
            cps[1 - cur] = pltpu.async_copy(
                yg.at[dref.at[c + 1]], bufs[1 - cur], sems[1 - cur])
        tok0 = wid * TPW + c * CCH
        pltpu.sync_copy(shared.at[pl.ds(tok0, CCH)], shbuf)
        cps[cur].wait()
        buf = bufs[cur]
        for tt in range(CCH):
            def body_k(k, carry):
                sl = pl.ds(k * 16, 16)
                acc = shbuf[tt, sl]
                for j in range(K_ROUTED):
                    acc = acc + buf[tt * K_ROUTED + j, sl]
                obuf[tt, sl] = acc
                return carry
            lax.fori_loop(0, D // 16, body_k, 0)
        pltpu.sync_copy(obuf, y.at[pl.ds(tok0, CCH)])


def _combine_sc(yg_sorted, desti2, shared_out):
    mesh = plsc.VectorSubcoreMesh(core_axis_name="c", subcore_axis_name="s")
    f = pl.kernel(
        _combine_sc_body,
        mesh=mesh,
        out_type=jax.ShapeDtypeStruct((T, D), jnp.float32),
        scratch_types=[
            pltpu.VMEM((_NCC, CCH * K_ROUTED), jnp.int32),
            pltpu.VMEM((CCH * K_ROUTED, D), jnp.float32),
            pltpu.VMEM((CCH * K_ROUTED, D), jnp.float32),
            pltpu.VMEM((CCH, D), jnp.float32),
            pltpu.VMEM((CCH, D), jnp.float32),
            pltpu.SemaphoreType.DMA,
            pltpu.SemaphoreType.DMA,
        ],
    )
    return f(yg_sorted, desti2, shared_out)


def _dispatch_jnp(x_pad, row_flat, dest_flat, g_flat):
    rows_sorted = jnp.full((PADROWS,), T, jnp.int32).at[dest_flat].set(row_flat)
    x_sorted = x_pad[rows_sorted]
    g_sorted = jnp.zeros((PADROWS, 16), jnp.float32).at[dest_flat, 0].set(g_flat)
    return x_sorted, g_sorted


def _combine_jnp(yg_sorted, dest6, shared_out):
    contrib = yg_sorted[dest6.reshape(-1)].reshape(T, K_ROUTED, D)
    return shared_out + jnp.sum(contrib, axis=1)


def _moe(x, Wg, expert_bias, Ws1, Ws2, Ws3, Wr1, Wr2, Wr3, interpret=False):
    x_flat = x.reshape(T, D)
    g8, dest8, counts8 = _router(x_flat, Wg, expert_bias, interpret=interpret)

    counts = counts8[0, :N_ROUTED]
    nb = (counts + (BLK - 1)) // BLK
    cumb = jnp.cumsum(nb)
    bidx = jnp.arange(NB, dtype=jnp.int32)
    block_expert = jnp.minimum(
        jnp.sum(cumb[None, :] <= bidx[:, None], axis=1), N_ROUTED - 1
    ).astype(jnp.int32)
    pstart = (cumb - nb) * BLK
    bexp_limit = pstart[block_expert] + counts[block_expert] - bidx * BLK
    block_limit = jnp.clip(bexp_limit, 0, BLK).astype(jnp.int32)

    dest6 = dest8[:, :K_ROUTED]
    x_pad = jnp.concatenate([x_flat, jnp.zeros((1, D), jnp.float32)], axis=0)

    if interpret:
        dest_flat = dest6.reshape(-1)
        row_flat = jnp.repeat(jnp.arange(T, dtype=jnp.int32), K_ROUTED)
        g_flat = g8[:, :K_ROUTED].reshape(-1)
        x_sorted, g_sorted = _dispatch_jnp(x_pad, row_flat, dest_flat, g_flat)
        shared_out = _shared(x_flat, Ws1, Ws2, Ws3, interpret=interpret)
        yg_sorted = _gmm(x_sorted, g_sorted, Wr1, Wr2, Wr3,
                         block_expert, block_limit, interpret=interpret)
        y = _combine_jnp(yg_sorted, dest6, shared_out)
        return y.reshape(B, T, D), jnp.asarray(0.0, dtype=jnp.float32)

    desti = dest6.reshape(NW, TPW, K_ROUTED).transpose(0, 2, 1)
    g_col = g8[:, :K_ROUTED].reshape(NW, TPW, K_ROUTED).transpose(0, 2, 1)
    gw = jnp.zeros((NW, K_ROUTED, TPW, 128), jnp.float32).at[..., 0].set(g_col)
    x_sorted, g_sorted = _dispatch_sc(x_flat, desti, gw)
    shared_out = _shared(x_flat, Ws1, Ws2, Ws3, interpret=interpret)
    yg_sorted = _gmm(x_sorted, g_sorted, Wr1, Wr2, Wr3,
                     block_expert, block_limit, interpret=interpret)
    desti2 = dest6.reshape(NW, TPW // CCH, CCH * K_ROUTED)
    y = _combine_sc(yg_sorted, desti2, shared_out)
    aux_loss = jnp.asarray(0.0, dtype=jnp.float32)
    return y.reshape(B, T, D), aux_loss


def kernel(x, Wg, expert_bias, Ws1, Ws2, Ws3, Wr1, Wr2, Wr3):
    return _moe(x, Wg, expert_bias, Ws1, Ws2, Ws3, Wr1, Wr2, Wr3)


# speedup vs baseline: 1.7297x; 1.0116x over previous
import jax
import jax.numpy as jnp
from jax import lax
from jax.experimental import pallas as pl
from jax.experimental.pallas import tpu as pltpu
from jax.experimental.pallas import tpu_sc as plsc

B, T, D = 1, 2048, 1024
N_EXP, N_SHARED = 64, 2
N_ROUTED = N_EXP - N_SHARED
K_ROUTED = 6
D_FF = 512
E_PAD = 64
BLK = 128
NB = 160
PADROWS = NB * BLK
N_ASSIGN = T * K_ROUTED

NW = 32
TPW = T // NW
CCH = 8
_NCC = TPW // CCH

_NEG = -1e30


def _router_body(x_ref, wgt_ref, bias_ref, tri_ref, up_ref,
                 g_ref, dest_ref, counts_ref):
    x = x_ref[...]
    logits = jnp.dot(x, wgt_ref[...], preferred_element_type=jnp.float32)
    s = 1.0 / (1.0 + jnp.exp(-logits))
    col = lax.broadcasted_iota(jnp.int32, (T, E_PAD), 1).astype(jnp.float32)
    valid_e = col < float(N_ROUTED)
    bias = bias_ref[0:1, :]
    biased = jnp.where(valid_e, s + bias, _NEG)

    M = jnp.zeros((T, E_PAD), jnp.float32)
    sels = []
    ohs = []
    for _ in range(K_ROUTED):
        m = jnp.max(biased, axis=1, keepdims=True)
        is_max = biased >= m
        idx = jnp.min(jnp.where(is_max, col, float(E_PAD)), axis=1,
                      keepdims=True)
        oh = (col == idx).astype(jnp.float32)
        sels.append(jnp.sum(s * oh, axis=1, keepdims=True))
        ohs.append(oh)
        biased = jnp.where(oh > 0.0, _NEG, biased)
        M = M + oh

    sel = jnp.concatenate(sels, axis=1)
    g = sel / (jnp.sum(sel, axis=1, keepdims=True) + 1e-20)

    cum = jnp.dot(tri_ref[...], M.astype(jnp.bfloat16),
                  preferred_element_type=jnp.float32)
    counts = jnp.sum(M, axis=0, keepdims=True)
    nb = jnp.floor((counts + float(BLK - 1)) * (1.0 / BLK))
    pstart = float(BLK) * jnp.dot(nb, up_ref[...],
                                  preferred_element_type=jnp.float32)

    base = pstart + cum
    dests = [jnp.sum(ohs[j] * base, axis=1, keepdims=True)
             for j in range(K_ROUTED)]
    dest = jnp.concatenate(dests, axis=1)

    zeros2 = jnp.zeros((T, 2), jnp.float32)
    g_ref[...] = jnp.concatenate([g, zeros2], axis=1)
    dest_ref[...] = jnp.concatenate([dest, zeros2], axis=1).astype(jnp.int32)
    counts_ref[...] = jnp.broadcast_to(counts, (8, E_PAD)).astype(jnp.int32)


def _router(x_flat, Wg, expert_bias):
    wgt = jnp.zeros((D, E_PAD), jnp.float32).at[:, :N_ROUTED].set(Wg.T)
    bias = jnp.zeros((8, E_PAD), jnp.float32).at[:, :N_ROUTED].set(
        expert_bias[None, :])
    tri = jnp.tril(jnp.ones((T, T), jnp.bfloat16), -1)
    up = jnp.triu(jnp.ones((E_PAD, E_PAD), jnp.float32), 1)
    return pl.pallas_call(
        _router_body,
        out_shape=(
            jax.ShapeDtypeStruct((T, 8), jnp.float32),
            jax.ShapeDtypeStruct((T, 8), jnp.int32),
            jax.ShapeDtypeStruct((8, E_PAD), jnp.int32),
        ),
    )(x_flat, wgt, bias, tri, up)


def _shared_body(x_ref, w1_ref, w3_ref, w2_ref, o_ref):
    x = x_ref[...].astype(jnp.bfloat16)
    acc = jnp.zeros((x.shape[0], D), jnp.float32)
    for i in range(N_SHARED):
        h1 = jnp.dot(x, w1_ref[i].astype(jnp.bfloat16).T,
                     preferred_element_type=jnp.float32)
        h3 = jnp.dot(x, w3_ref[i].astype(jnp.bfloat16).T,
                     preferred_element_type=jnp.float32)
        h = (h1 * (1.0 / (1.0 + jnp.exp(-h1))) * h3).astype(jnp.bfloat16)
        acc = acc + jnp.dot(h, w2_ref[i].astype(jnp.bfloat16).T,
                            preferred_element_type=jnp.float32)
    o_ref[...] = acc


def _shared(x_flat, Ws1, Ws2, Ws3):
    blk = 512
    return pl.pallas_call(
        _shared_body,
        grid=(T // blk,),
        in_specs=[
            pl.BlockSpec((blk, D), lambda i: (i, 0)),
            pl.BlockSpec((N_SHARED, D_FF, D), lambda i: (0, 0, 0)),
            pl.BlockSpec((N_SHARED, D_FF, D), lambda i: (0, 0, 0)),
            pl.BlockSpec((N_SHARED, D, D_FF), lambda i: (0, 0, 0)),
        ],
        out_specs=pl.BlockSpec((blk, D), lambda i: (i, 0)),
        out_shape=jax.ShapeDtypeStruct((T, D), jnp.float32),
    )(x_flat, Ws1, Ws3, Ws2)


def _gmm_body(be_ref, bl_ref, x_ref, g_ref, w1_ref, w3_ref, w2_ref, o_ref):
    b = pl.program_id(0)
    limit = bl_ref[b]

    @pl.when(limit > 0)
    def _():
        x = x_ref[...].astype(jnp.bfloat16)
        h1 = jnp.dot(x, w1_ref[0].astype(jnp.bfloat16).T,
                     preferred_element_type=jnp.float32)
        h3 = jnp.dot(x, w3_ref[0].astype(jnp.bfloat16).T,
                     preferred_element_type=jnp.float32)
        h = (h1 * (1.0 / (1.0 + jnp.exp(-h1))) * h3).astype(jnp.bfloat16)
        y = jnp.dot(h, w2_ref[0].astype(jnp.bfloat16).T,
                    preferred_element_type=jnp.float32)
        rows = lax.broadcasted_iota(jnp.int32, (BLK, 1), 0)
        gval = jnp.where(rows < limit, g_ref[:, 0:1], 0.0)
        o_ref[...] = y * gval

    @pl.when(limit <= 0)
    def _():
        o_ref[...] = jnp.zeros((BLK, D), jnp.float32)


def _gmm(x_sorted, g_sorted, Wr1, Wr2, Wr3, block_expert, block_limit):
    def _in_idx(b, be, bl):
        return (jnp.where(bl[b] > 0, b, 0), 0)

    def _w_idx(b, be, bl):
        return (be[b], 0, 0)

    def _out_idx(b, be, bl):
        return (jnp.where(bl[b] > 0, b, NB - 1), 0)

    grid_spec = pltpu.PrefetchScalarGridSpec(
        num_scalar_prefetch=2,
        grid=(NB,),
        in_specs=[
            pl.BlockSpec((BLK, D), _in_idx),
            pl.BlockSpec((BLK, 128), _in_idx),
            pl.BlockSpec((1, D_FF, D), _w_idx),
            pl.BlockSpec((1, D_FF, D), _w_idx),
            pl.BlockSpec((1, D, D_FF), _w_idx),
        ],
        out_specs=pl.BlockSpec((BLK, D), _out_idx),
    )
    return pl.pallas_call(
        _gmm_body,
        grid_spec=grid_spec,
        out_shape=jax.ShapeDtypeStruct((PADROWS, D), jnp.float32),
        compiler_params=pltpu.CompilerParams(
            dimension_semantics=("arbitrary",)),
    )(block_expert, block_limit, x_sorted, g_sorted, Wr1, Wr3, Wr2)


def _dispatch_sc_body(xflat, desti, gw, xs, gs, didx, buf, gwb, sem, sem2):
    wid = lax.axis_index("s") * 2 + lax.axis_index("c")
    tok0 = wid * TPW
    pltpu.sync_copy(desti.at[wid], didx)
    pltpu.sync_copy(gw.at[wid], gwb)
    pltpu.sync_copy(xflat.at[pl.ds(tok0, TPW)], buf)
    cps = []
    for j in range(K_ROUTED):
        cps.append(pltpu.async_copy(buf, xs.at[didx.at[j]], sem))
        cps.append(pltpu.async_copy(gwb.at[j], gs.at[didx.at[j]], sem2))
    for cp in cps:
        cp.wait()


def _dispatch_sc(x_flat, desti, gw):
    mesh = plsc.VectorSubcoreMesh(core_axis_name="c", subcore_axis_name="s")
    f = pl.kernel(
        _dispatch_sc_body,
        mesh=mesh,
        out_type=(
            jax.ShapeDtypeStruct((PADROWS, D), jnp.float32),
            jax.ShapeDtypeStruct((PADROWS, 128), jnp.float32),
        ),
        scratch_types=[
            pltpu.VMEM((K_ROUTED, TPW), jnp.int32),
            pltpu.VMEM((TPW, D), jnp.float32),
            pltpu.VMEM((K_ROUTED, TPW, 128), jnp.float32),
            pltpu.SemaphoreType.DMA,
            pltpu.SemaphoreType.DMA,
        ],
    )
    return f(x_flat, desti, gw)


def _combine_sc_body(yg, desti, shared, y, dref,
                     buf0, buf1, shbuf, obuf, sem0, sem1):
    wid = lax.axis_index("s") * 2 + lax.axis_index("c")
    pltpu.sync_copy(desti.at[wid], dref)
    bufs = (buf0, buf1)
    sems = (sem0, sem1)
    cps = [None, None]
    cps[0] = pltpu.async_copy(yg.at[dref.at[0]], bufs[0], sems[0])
    for c in range(_NCC):
        cur = c % 2
        if c + 1 < _NCC:
            cps[1 - cur] = pltpu.async_copy(
                yg.at[dref.at[c + 1]], bufs[1 - cur], sems[1 - cur])
        tok0 = wid * TPW + c * CCH
        pltpu.sync_copy(shared.at[pl.ds(tok0, CCH)], shbuf)
        cps[cur].wait()
        buf = bufs[cur]
        for tt in range(CCH):
            def body_k(k, carry):
                sl = pl.ds(k * 16, 16)
                acc = shbuf[tt, sl]
                for j in range(K_ROUTED):
                    acc = acc + buf[tt * K_ROUTED + j, sl]
                obuf[tt, sl] = acc
                return carry
            lax.fori_loop(0, D // 16, body_k, 0)
        pltpu.sync_copy(obuf, y.at[pl.ds(tok0, CCH)])


def _combine_sc(yg_sorted, desti2, shared_out):
    mesh = plsc.VectorSubcoreMesh(core_axis_name="c", subcore_axis_name="s")
    f = pl.kernel(
        _combine_sc_body,
        mesh=mesh,
        out_type=jax.ShapeDtypeStruct((T, D), jnp.float32),
        scratch_types=[
            pltpu.VMEM((_NCC, CCH * K_ROUTED), jnp.int32),
            pltpu.VMEM((CCH * K_ROUTED, D), jnp.float32),
            pltpu.VMEM((CCH * K_ROUTED, D), jnp.float32),
            pltpu.VMEM((CCH, D), jnp.float32),
            pltpu.VMEM((CCH, D), jnp.float32),
            pltpu.SemaphoreType.DMA,
            pltpu.SemaphoreType.DMA,
        ],
    )
    return f(yg_sorted, desti2, shared_out)


def kernel(x, Wg, expert_bias, Ws1, Ws2, Ws3, Wr1, Wr2, Wr3):
    x_flat = x.reshape(T, D)
    g8, dest8, counts8 = _router(x_flat, Wg, expert_bias)

    counts = counts8[0, :N_ROUTED]
    nb = (counts + (BLK - 1)) // BLK
    cumb = jnp.cumsum(nb)
    bidx = jnp.arange(NB, dtype=jnp.int32)
    block_expert = jnp.minimum(
        jnp.sum(cumb[None, :] <= bidx[:, None], axis=1), N_ROUTED - 1
    ).astype(jnp.int32)
    pstart = (cumb - nb) * BLK
    bexp_limit = pstart[block_expert] + counts[block_expert] - bidx * BLK
    block_limit = jnp.clip(bexp_limit, 0, BLK).astype(jnp.int32)

    dest6 = dest8[:, :K_ROUTED]
    desti = dest6.reshape(NW, TPW, K_ROUTED).transpose(0, 2, 1)
    g_col = g8[:, :K_ROUTED].reshape(NW, TPW, K_ROUTED).transpose(0, 2, 1)
    gw = jnp.zeros((NW, K_ROUTED, TPW, 128), jnp.float32).at[..., 0].set(g_col)
    x_sorted, g_sorted = _dispatch_sc(x_flat, desti, gw)
    shared_out = _shared(x_flat, Ws1, Ws2, Ws3)
    yg_sorted = _gmm(x_sorted, g_sorted, Wr1, Wr2, Wr3,
                     block_expert, block_limit)
    desti2 = dest6.reshape(NW, _NCC, CCH * K_ROUTED)
    y = _combine_sc(yg_sorted, desti2, shared_out)
    aux_loss = jnp.asarray(0.0, dtype=jnp.float32)
    return y.reshape(B, T, D), aux_loss
